# baseline (speedup 1.0000x reference)
import functools

import jax
import jax.numpy as jnp
from jax import lax
from jax.experimental import pallas as pl
from jax.experimental.pallas import tpu as pltpu
from jax.experimental.pallas import tpu_sc as plsc

H = 32
N = 10000
E = 320000
F = 128

NC = 2
NS = 16
NW = NC * NS

SUB = 80
NSUB = 5
CH = SUB * NSUB
EPW = E // NW
NCHUNK = EPW // CH
RPT = 624
TAIL = N - NS * RPT

_f32 = jnp.float32


def _sc_edge_kernel():
    mesh = plsc.VectorSubcoreMesh(core_axis_name="c", subcore_axis_name="s")

    @functools.partial(
        pl.kernel,
        mesh=mesh,
        compiler_params=pltpu.CompilerParams(use_tc_tiling_on_sc=False),
        out_type=[
            jax.ShapeDtypeStruct((NC, N, H), _f32),
            jax.ShapeDtypeStruct((NC, N, 1), _f32),
        ],
        scratch_types=[
            pltpu.VMEM((2, NSUB, SUB), jnp.int32),
            pltpu.VMEM((2, NSUB, SUB), jnp.int32),
            pltpu.VMEM((2, NSUB, SUB * 4), _f32),
            pltpu.VMEM((2, NSUB, SUB, H), _f32),
            pltpu.VMEM((2, NSUB, SUB, H), _f32),
            pltpu.VMEM((2, NSUB, SUB, H), _f32),
            pltpu.VMEM((SUB, 1), _f32),
            pltpu.VMEM((8, H), _f32),
            pltpu.VMEM_SHARED((N, H), _f32),
            pltpu.VMEM_SHARED((N, 1), _f32),
            pltpu.SemaphoreType.DMA((2,)),
            pltpu.SemaphoreType.DMA((2,)),
        ],
    )
    def k(p_h, q_h, src_h, dst_h, ef_h, cw_h, ones_h, zrow_h, zcol_h,
          agg_o, deg_o,
          srcv, dstv, efv, pv, qv, sv, onesv, cwv, aggS, degS, gsem, ssem):
        cid = lax.axis_index("c")
        sid = lax.axis_index("s")
        wid = cid * NS + sid

        pltpu.sync_copy(cw_h, cwv)
        pltpu.sync_copy(ones_h, onesv)
        pltpu.sync_copy(zrow_h.at[pl.ds(0, RPT)], aggS.at[pl.ds(sid * RPT, RPT)])
        pltpu.sync_copy(zcol_h.at[pl.ds(0, RPT)], degS.at[pl.ds(sid * RPT, RPT)])

        @pl.when(sid == NS - 1)
        def _():
            pltpu.sync_copy(zrow_h.at[pl.ds(0, TAIL)],
                            aggS.at[pl.ds(NS * RPT, TAIL)])
            pltpu.sync_copy(zcol_h.at[pl.ds(0, TAIL)],
                            degS.at[pl.ds(NS * RPT, TAIL)])

        plsc.subcore_barrier()

        cw = [[cwv[j, 0:16], cwv[j, 16:32]] for j in range(3)]

        def fire(g, b):
            j = wid * NCHUNK + g
            pltpu.sync_copy(src_h.at[j], srcv.at[b])
            pltpu.sync_copy(dst_h.at[j], dstv.at[b])
            pltpu.sync_copy(ef_h.at[j], efv.at[b])
            for c in range(NSUB):
                pltpu.async_copy(p_h.at[srcv.at[b, c]], pv.at[b, c], gsem.at[b])
                pltpu.async_copy(q_h.at[dstv.at[b, c]], qv.at[b, c], gsem.at[b])

        def wait_gathers(b):
            for c in range(NSUB):
                pltpu.make_async_copy(p_h.at[srcv.at[b, c]], pv.at[b, c],
                                      gsem.at[b]).wait()
                pltpu.make_async_copy(q_h.at[dstv.at[b, c]], qv.at[b, c],
                                      gsem.at[b]).wait()

        def wait_scatters(b):
            for c in range(NSUB):
                pltpu.make_async_copy(sv.at[b, c], aggS.at[dstv.at[b, c]],
                                      ssem.at[b]).wait()
                pltpu.make_async_copy(onesv, degS.at[dstv.at[b, c]],
                                      ssem.at[b]).wait()

        fire(0, 0)

        def chunk_body(g, carry):
            b = g % 2
            nb = (g + 1) % 2

            @pl.when(g + 1 < NCHUNK)
            def _():
                fire(g + 1, nb)

            wait_gathers(b)

            for c in range(NSUB):
                def quad_body(e4, _, c=c):
                    efq = efv[b, c, pl.ds(e4 * 16, 16)]
                    for kk in range(4):
                        e = e4 * 4 + kk
                        f0 = efq[4 * kk]
                        f1 = efq[4 * kk + 1]
                        f2 = efq[4 * kk + 2]
                        t0 = f0 * cw[0][0] + f1 * cw[1][0] + f2 * cw[2][0]
                        t1 = f0 * cw[0][1] + f1 * cw[1][1] + f2 * cw[2][1]
                        a0 = pv[b, c, e, 0:16] + qv[b, c, e, 0:16] + t0
                        a1 = pv[b, c, e, 16:32] + qv[b, c, e, 16:32] + t1
                        sv[b, c, e, 0:16] = jnp.maximum(a0, 0.0)
                        sv[b, c, e, 16:32] = jnp.maximum(a1, 0.0)
                    return 0

                lax.fori_loop(0, SUB // 4, quad_body, 0)

            for c in range(NSUB):
                pltpu.sync_copy(sv.at[b, c], aggS.at[dstv.at[b, c]], add=True)
                pltpu.sync_copy(onesv, degS.at[dstv.at[b, c]], add=True)
            return carry

        lax.fori_loop(0, NCHUNK, chunk_body, 0)

        plsc.subcore_barrier()
        pltpu.sync_copy(aggS.at[pl.ds(sid * RPT, RPT)],
                        agg_o.at[cid, pl.ds(sid * RPT, RPT)])
        pltpu.sync_copy(degS.at[pl.ds(sid * RPT, RPT)],
                        deg_o.at[cid, pl.ds(sid * RPT, RPT)])

        @pl.when(sid == NS - 1)
        def _():
            pltpu.sync_copy(aggS.at[pl.ds(NS * RPT, TAIL)],
                            agg_o.at[cid, pl.ds(NS * RPT, TAIL)])
            pltpu.sync_copy(degS.at[pl.ds(NS * RPT, TAIL)],
                            deg_o.at[cid, pl.ds(NS * RPT, TAIL)])

    return k


_SC_EDGE = _sc_edge_kernel()


def _tc_enc_body(nf, we1, be1, we2, be2, w1s, w1d, b1, h_o, p_o, q_o):
    t = jnp.maximum(jnp.dot(nf[...], we1[...], preferred_element_type=_f32)
                    + be1[...], 0.0)
    h = jnp.dot(t, we2[...], preferred_element_type=_f32) + be2[...]
    h_o[...] = h
    p_o[...] = jnp.dot(h, w1s[...], preferred_element_type=_f32) + b1[...]
    q_o[...] = jnp.dot(h, w1d[...], preferred_element_type=_f32)


@jax.jit
def _tc_enc(nf, we1, be1, we2, be2, w1s, w1d, b1):
    return pl.pallas_call(
        _tc_enc_body,
        out_shape=[jax.ShapeDtypeStruct((N, H), _f32)] * 3,
    )(nf, we1, be1, we2, be2, w1s, w1d, b1)


def _tc_upd_body(has_next, h, ag, dg, w2, b2, wua, wub, bu1, wu2, bu2, *rest):
    aggp = ag[0] + ag[1]
    deg = dg[0] + dg[1]
    agg = jnp.dot(aggp, w2[...], preferred_element_type=_f32) + deg * b2[...]
    t = jnp.maximum(jnp.dot(h[...], wua[...], preferred_element_type=_f32)
                    + jnp.dot(agg, wub[...], preferred_element_type=_f32)
                    + bu1[...], 0.0)
    hn = h[...] + jnp.dot(t, wu2[...], preferred_element_type=_f32) + bu2[...]
    if has_next:
        w1s, w1d, b1, h_o, p_o, q_o = rest
        h_o[...] = hn
        p_o[...] = jnp.dot(hn, w1s[...], preferred_element_type=_f32) + b1[...]
        q_o[...] = jnp.dot(hn, w1d[...], preferred_element_type=_f32)
    else:
        wh1, bh1, wh2, bh2, out_o = rest
        u = jnp.maximum(jnp.dot(hn, wh1[...], preferred_element_type=_f32)
                        + bh1[...], 0.0)
        out_o[...] = jnp.dot(u, wh2[...], preferred_element_type=_f32) + bh2[...]


def _tc_upd(has_next, h, ag, dg, w2, b2, wua, wub, bu1, wu2, bu2, *rest):
    if has_next:
        out_shape = [jax.ShapeDtypeStruct((N, H), _f32)] * 3
    else:
        out_shape = [jax.ShapeDtypeStruct((N, 1), _f32)]
    return pl.pallas_call(
        functools.partial(_tc_upd_body, has_next),
        out_shape=out_shape,
    )(h, ag, dg, w2, b2, wua, wub, bu1, wu2, bu2, *rest)


def _row(b):
    return b.reshape(1, -1)


def kernel(node_features, edges, edge_features, params):
    src = edges[:, 0].reshape(NW * NCHUNK, NSUB, SUB)
    dst = edges[:, 1].reshape(NW * NCHUNK, NSUB, SUB)
    ef = jnp.pad(edge_features, ((0, 0), (0, 1))).reshape(NW * NCHUNK, NSUB, SUB * 4)

    ones80 = jnp.ones((SUB, 1), _f32)
    zrow = jnp.zeros((RPT, H), _f32)
    zcol = jnp.zeros((RPT, 1), _f32)

    enc, msg, upd, head = params["enc"], params["msg"], params["upd"], params["head"]

    msg_parts = []
    for r in range(2):
        w1 = msg[r][0]["W"]
        cw = jnp.zeros((8, H), _f32).at[0:3].set(w1[2 * H:2 * H + 3])
        msg_parts.append({
            "w1s": w1[0:H], "w1d": w1[H:2 * H], "cw": cw,
            "b1": _row(msg[r][0]["b"]),
            "w2": msg[r][1]["W"], "b2": _row(msg[r][1]["b"]),
        })

    h, p, q = _tc_enc(node_features, enc[0]["W"], _row(enc[0]["b"]),
                      enc[1]["W"], _row(enc[1]["b"]),
                      msg_parts[0]["w1s"], msg_parts[0]["w1d"],
                      msg_parts[0]["b1"])

    for r in range(2):
        mp = msg_parts[r]
        ag, dg = _SC_EDGE(p, q, src, dst, ef, mp["cw"], ones80, zrow, zcol)
        uw = upd[r][0]["W"]
        args = (h, ag, dg, mp["w2"], mp["b2"],
                uw[0:H], uw[H:2 * H], _row(upd[r][0]["b"]),
                upd[r][1]["W"], _row(upd[r][1]["b"]))
        if r == 0:
            nxt = msg_parts[1]
            h, p, q = _tc_upd(True, *args, nxt["w1s"], nxt["w1d"], nxt["b1"])
        else:
            out, = _tc_upd(False, *args, head[0]["W"], _row(head[0]["b"]),
                           head[1]["W"], _row(head[1]["b"]))
    return out[:, 0]


# speedup vs baseline: 1.0870x; 1.0870x over previous
import functools

import jax
import jax.numpy as jnp
from jax import lax
from jax.experimental import pallas as pl
from jax.experimental.pallas import tpu as pltpu
from jax.experimental.pallas import tpu_sc as plsc

H = 32
N = 10000
E = 320000
F = 128

NC = 2
NS = 16
NW = NC * NS

SUB = 80
NSUB = 5
CH = SUB * NSUB
EPW = E // NW
NCHUNK = EPW // CH
RPT = 624
TAIL = N - NS * RPT

_f32 = jnp.float32


def _sc_edge_kernel():
    mesh = plsc.VectorSubcoreMesh(core_axis_name="c", subcore_axis_name="s")

    @functools.partial(
        pl.kernel,
        mesh=mesh,
        compiler_params=pltpu.CompilerParams(use_tc_tiling_on_sc=False),
        out_type=[
            jax.ShapeDtypeStruct((NC, N, H), _f32),
            jax.ShapeDtypeStruct((NC, N, 1), _f32),
        ],
        scratch_types=[
            pltpu.VMEM((NCHUNK, NSUB, SUB), jnp.int32),
            pltpu.VMEM((NCHUNK, NSUB, SUB), jnp.int32),
            pltpu.VMEM((NSUB, SUB * 4), _f32),
            pltpu.VMEM((NSUB, SUB, H), _f32),
            pltpu.VMEM((NSUB, SUB, H), _f32),
            pltpu.VMEM((NSUB, SUB, H), _f32),
            pltpu.VMEM((SUB, 1), _f32),
            pltpu.VMEM((8, H), _f32),
            pltpu.VMEM_SHARED((N, H), _f32),
            pltpu.VMEM_SHARED((N, H), _f32),
            pltpu.VMEM_SHARED((N, H), _f32),
            pltpu.VMEM_SHARED((N, 1), _f32),
            pltpu.SemaphoreType.DMA,
        ],
    )
    def k(p_h, q_h, src_h, dst_h, ef_h, cw_h, ones_h, zrow_h, zcol_h,
          agg_o, deg_o,
          srcv, dstv, efv, pv, qv, sv, onesv, cwv, pS, qS, aggS, degS,
          gsem):
        cid = lax.axis_index("c")
        sid = lax.axis_index("s")
        wid = cid * NS + sid

        pltpu.sync_copy(cw_h, cwv)
        pltpu.sync_copy(ones_h, onesv)
        pltpu.sync_copy(src_h.at[wid], srcv)
        pltpu.sync_copy(dst_h.at[wid], dstv)
        pltpu.sync_copy(p_h.at[pl.ds(sid * RPT, RPT)], pS.at[pl.ds(sid * RPT, RPT)])
        pltpu.sync_copy(q_h.at[pl.ds(sid * RPT, RPT)], qS.at[pl.ds(sid * RPT, RPT)])
        pltpu.sync_copy(zrow_h.at[pl.ds(0, RPT)], aggS.at[pl.ds(sid * RPT, RPT)])
        pltpu.sync_copy(zcol_h.at[pl.ds(0, RPT)], degS.at[pl.ds(sid * RPT, RPT)])

        @pl.when(sid == NS - 1)
        def _():
            pltpu.sync_copy(p_h.at[pl.ds(NS * RPT, TAIL)],
                            pS.at[pl.ds(NS * RPT, TAIL)])
            pltpu.sync_copy(q_h.at[pl.ds(NS * RPT, TAIL)],
                            qS.at[pl.ds(NS * RPT, TAIL)])
            pltpu.sync_copy(zrow_h.at[pl.ds(0, TAIL)],
                            aggS.at[pl.ds(NS * RPT, TAIL)])
            pltpu.sync_copy(zcol_h.at[pl.ds(0, TAIL)],
                            degS.at[pl.ds(NS * RPT, TAIL)])

        plsc.subcore_barrier()

        cw = [[cwv[j, 0:16], cwv[j, 16:32]] for j in range(3)]

        def chunk_body(g, carry):
            pltpu.sync_copy(ef_h.at[wid, g], efv)
            cps = []
            for c in range(NSUB):
                cps.append(pltpu.async_copy(pS.at[srcv.at[g, c]], pv.at[c], gsem))
                cps.append(pltpu.async_copy(qS.at[dstv.at[g, c]], qv.at[c], gsem))
            for cp in cps:
                cp.wait()

            for c in range(NSUB):
                def quad_body(e4, _, c=c):
                    efq = efv[c, pl.ds(e4 * 16, 16)]
                    for kk in range(4):
                        e = e4 * 4 + kk
                        f0 = efq[4 * kk]
                        f1 = efq[4 * kk + 1]
                        f2 = efq[4 * kk + 2]
                        t0 = f0 * cw[0][0] + f1 * cw[1][0] + f2 * cw[2][0]
                        t1 = f0 * cw[0][1] + f1 * cw[1][1] + f2 * cw[2][1]
                        a0 = pv[c, e, 0:16] + qv[c, e, 0:16] + t0
                        a1 = pv[c, e, 16:32] + qv[c, e, 16:32] + t1
                        sv[c, e, 0:16] = jnp.maximum(a0, 0.0)
                        sv[c, e, 16:32] = jnp.maximum(a1, 0.0)
                    return 0

                lax.fori_loop(0, SUB // 4, quad_body, 0)

            for c in range(NSUB):
                pltpu.sync_copy(sv.at[c], aggS.at[dstv.at[g, c]], add=True)
                pltpu.sync_copy(onesv, degS.at[dstv.at[g, c]], add=True)
            return carry

        lax.fori_loop(0, NCHUNK, chunk_body, 0)

        plsc.subcore_barrier()
        pltpu.sync_copy(aggS.at[pl.ds(sid * RPT, RPT)],
                        agg_o.at[cid, pl.ds(sid * RPT, RPT)])
        pltpu.sync_copy(degS.at[pl.ds(sid * RPT, RPT)],
                        deg_o.at[cid, pl.ds(sid * RPT, RPT)])

        @pl.when(sid == NS - 1)
        def _():
            pltpu.sync_copy(aggS.at[pl.ds(NS * RPT, TAIL)],
                            agg_o.at[cid, pl.ds(NS * RPT, TAIL)])
            pltpu.sync_copy(degS.at[pl.ds(NS * RPT, TAIL)],
                            deg_o.at[cid, pl.ds(NS * RPT, TAIL)])

    return k


_SC_EDGE = _sc_edge_kernel()


def _tc_enc_body(nf, we1, be1, we2, be2, w1s, w1d, b1, h_o, p_o, q_o):
    t = jnp.maximum(jnp.dot(nf[...], we1[...], preferred_element_type=_f32)
                    + be1[...], 0.0)
    h = jnp.dot(t, we2[...], preferred_element_type=_f32) + be2[...]
    h_o[...] = h
    p_o[...] = jnp.dot(h, w1s[...], preferred_element_type=_f32) + b1[...]
    q_o[...] = jnp.dot(h, w1d[...], preferred_element_type=_f32)


@jax.jit
def _tc_enc(nf, we1, be1, we2, be2, w1s, w1d, b1):
    return pl.pallas_call(
        _tc_enc_body,
        out_shape=[jax.ShapeDtypeStruct((N, H), _f32)] * 3,
    )(nf, we1, be1, we2, be2, w1s, w1d, b1)


def _tc_upd_body(has_next, h, ag, dg, w2, b2, wua, wub, bu1, wu2, bu2, *rest):
    aggp = ag[0] + ag[1]
    deg = dg[0] + dg[1]
    agg = jnp.dot(aggp, w2[...], preferred_element_type=_f32) + deg * b2[...]
    t = jnp.maximum(jnp.dot(h[...], wua[...], preferred_element_type=_f32)
                    + jnp.dot(agg, wub[...], preferred_element_type=_f32)
                    + bu1[...], 0.0)
    hn = h[...] + jnp.dot(t, wu2[...], preferred_element_type=_f32) + bu2[...]
    if has_next:
        w1s, w1d, b1, h_o, p_o, q_o = rest
        h_o[...] = hn
        p_o[...] = jnp.dot(hn, w1s[...], preferred_element_type=_f32) + b1[...]
        q_o[...] = jnp.dot(hn, w1d[...], preferred_element_type=_f32)
    else:
        wh1, bh1, wh2, bh2, out_o = rest
        u = jnp.maximum(jnp.dot(hn, wh1[...], preferred_element_type=_f32)
                        + bh1[...], 0.0)
        out_o[...] = jnp.dot(u, wh2[...], preferred_element_type=_f32) + bh2[...]


def _tc_upd(has_next, h, ag, dg, w2, b2, wua, wub, bu1, wu2, bu2, *rest):
    if has_next:
        out_shape = [jax.ShapeDtypeStruct((N, H), _f32)] * 3
    else:
        out_shape = [jax.ShapeDtypeStruct((N, 1), _f32)]
    return pl.pallas_call(
        functools.partial(_tc_upd_body, has_next),
        out_shape=out_shape,
    )(h, ag, dg, w2, b2, wua, wub, bu1, wu2, bu2, *rest)


def _row(b):
    return b.reshape(1, -1)


def kernel(node_features, edges, edge_features, params):
    src = edges[:, 0].reshape(NW, NCHUNK, NSUB, SUB)
    dst = edges[:, 1].reshape(NW, NCHUNK, NSUB, SUB)
    ef = jnp.pad(edge_features, ((0, 0), (0, 1))).reshape(NW, NCHUNK, NSUB, SUB * 4)

    ones80 = jnp.ones((SUB, 1), _f32)
    zrow = jnp.zeros((RPT, H), _f32)
    zcol = jnp.zeros((RPT, 1), _f32)

    enc, msg, upd, head = params["enc"], params["msg"], params["upd"], params["head"]

    msg_parts = []
    for r in range(2):
        w1 = msg[r][0]["W"]
        cw = jnp.zeros((8, H), _f32).at[0:3].set(w1[2 * H:2 * H + 3])
        msg_parts.append({
            "w1s": w1[0:H], "w1d": w1[H:2 * H], "cw": cw,
            "b1": _row(msg[r][0]["b"]),
            "w2": msg[r][1]["W"], "b2": _row(msg[r][1]["b"]),
        })

    h, p, q = _tc_enc(node_features, enc[0]["W"], _row(enc[0]["b"]),
                      enc[1]["W"], _row(enc[1]["b"]),
                      msg_parts[0]["w1s"], msg_parts[0]["w1d"],
                      msg_parts[0]["b1"])

    for r in range(2):
        mp = msg_parts[r]
        ag, dg = _SC_EDGE(p, q, src, dst, ef, mp["cw"], ones80, zrow, zcol)
        uw = upd[r][0]["W"]
        args = (h, ag, dg, mp["w2"], mp["b2"],
                uw[0:H], uw[H:2 * H], _row(upd[r][0]["b"]),
                upd[r][1]["W"], _row(upd[r][1]["b"]))
        if r == 0:
            nxt = msg_parts[1]
            h, p, q = _tc_upd(True, *args, nxt["w1s"], nxt["w1d"], nxt["b1"])
        else:
            out, = _tc_upd(False, *args, head[0]["W"], _row(head[0]["b"]),
                           head[1]["W"], _row(head[1]["b"]))
    return out[:, 0]


# speedup vs baseline: 1.8972x; 1.7454x over previous
import functools

import jax
import jax.numpy as jnp
from jax import lax
from jax.experimental import pallas as pl
from jax.experimental.pallas import tpu as pltpu
from jax.experimental.pallas import tpu_sc as plsc

H = 32
N = 10000
E = 320000
F = 128

NC = 2
NS = 16
NW = NC * NS

SUB = 80
NSUB = 5
CH = SUB * NSUB
EPW = E // NW
NCHUNK = EPW // CH
RPT = 624
TAIL = N - NS * RPT

_f32 = jnp.float32


def _sc_edge_kernel(with_deg):
    mesh = plsc.VectorSubcoreMesh(core_axis_name="c", subcore_axis_name="s")

    out_type = [jax.ShapeDtypeStruct((NC, N, H), _f32)]
    scratch = [
        pltpu.VMEM((NSUB, SUB), jnp.int32),
        pltpu.VMEM((NSUB, SUB), jnp.int32),
        pltpu.VMEM((4, EPW), _f32),
        pltpu.VMEM((NSUB, SUB, H), _f32),
        pltpu.VMEM((NSUB, SUB, H), _f32),
        pltpu.VMEM((NSUB, SUB, H), _f32),
        pltpu.VMEM((8, H), _f32),
        pltpu.VMEM_SHARED((N, H), _f32),
        pltpu.SemaphoreType.DMA,
    ]
    if with_deg:
        out_type.append(jax.ShapeDtypeStruct((NC, N, 1), _f32))
        scratch.insert(7, pltpu.VMEM((SUB, 1), _f32))
        scratch.insert(9, pltpu.VMEM_SHARED((N, 1), _f32))

    @functools.partial(
        pl.kernel,
        mesh=mesh,
        compiler_params=pltpu.CompilerParams(use_tc_tiling_on_sc=False),
        out_type=out_type,
        scratch_types=scratch,
    )
    def k(*refs):
        if with_deg:
            (p_h, q_h, src_h, dst_h, ef_h, cw_h, ones_h, zrow_h, zcol_h,
             agg_o, deg_o,
             srcv, dstv, efv, pv, qv, sv, cwv, onesv, aggS, degS, gsem) = refs
        else:
            (p_h, q_h, src_h, dst_h, ef_h, cw_h, zrow_h,
             agg_o,
             srcv, dstv, efv, pv, qv, sv, cwv, aggS, gsem) = refs
        cid = lax.axis_index("c")
        sid = lax.axis_index("s")
        wid = cid * NS + sid

        pltpu.sync_copy(cw_h, cwv)
        for j in range(3):
            pltpu.sync_copy(ef_h.at[j, pl.ds(wid * EPW, EPW)], efv.at[j])
        pltpu.sync_copy(zrow_h.at[pl.ds(0, RPT)], aggS.at[pl.ds(sid * RPT, RPT)])
        if with_deg:
            pltpu.sync_copy(ones_h, onesv)
            pltpu.sync_copy(zcol_h.at[pl.ds(0, RPT)],
                            degS.at[pl.ds(sid * RPT, RPT)])

        @pl.when(sid == NS - 1)
        def _():
            pltpu.sync_copy(zrow_h.at[pl.ds(0, TAIL)],
                            aggS.at[pl.ds(NS * RPT, TAIL)])
            if with_deg:
                pltpu.sync_copy(zcol_h.at[pl.ds(0, TAIL)],
                                degS.at[pl.ds(NS * RPT, TAIL)])

        plsc.subcore_barrier()

        cw = [[cwv[j, 0:16], cwv[j, 16:32]] for j in range(3)]

        def chunk_body(g, carry):
            pltpu.sync_copy(src_h.at[wid, g], srcv)
            pltpu.sync_copy(dst_h.at[wid, g], dstv)
            cps = []
            for c in range(NSUB):
                cps.append(pltpu.async_copy(p_h.at[srcv.at[c]], pv.at[c], gsem))
                cps.append(pltpu.async_copy(q_h.at[dstv.at[c]], qv.at[c], gsem))
            for cp in cps:
                cp.wait()

            for c in range(NSUB):
                def grp_body(e16, _, c=c):
                    off = g * CH + c * SUB + e16 * 16
                    f0v = efv[0, pl.ds(off, 16)]
                    f1v = efv[1, pl.ds(off, 16)]
                    f2v = efv[2, pl.ds(off, 16)]
                    for kk in range(16):
                        e = e16 * 16 + kk
                        t0 = (f0v[kk] * cw[0][0] + f1v[kk] * cw[1][0]
                              + f2v[kk] * cw[2][0])
                        t1 = (f0v[kk] * cw[0][1] + f1v[kk] * cw[1][1]
                              + f2v[kk] * cw[2][1])
                        a0 = pv[c, e, 0:16] + qv[c, e, 0:16] + t0
                        a1 = pv[c, e, 16:32] + qv[c, e, 16:32] + t1
                        sv[c, e, 0:16] = jnp.maximum(a0, 0.0)
                        sv[c, e, 16:32] = jnp.maximum(a1, 0.0)
                    return 0

                lax.fori_loop(0, SUB // 16, grp_body, 0)

            for c in range(NSUB):
                pltpu.sync_copy(sv.at[c], aggS.at[dstv.at[c]], add=True)
                if with_deg:
                    pltpu.sync_copy(onesv, degS.at[dstv.at[c]], add=True)
            return carry

        lax.fori_loop(0, NCHUNK, chunk_body, 0)

        plsc.subcore_barrier()
        pltpu.sync_copy(aggS.at[pl.ds(sid * RPT, RPT)],
                        agg_o.at[cid, pl.ds(sid * RPT, RPT)])
        if with_deg:
            pltpu.sync_copy(degS.at[pl.ds(sid * RPT, RPT)],
                            deg_o.at[cid, pl.ds(sid * RPT, RPT)])

        @pl.when(sid == NS - 1)
        def _():
            pltpu.sync_copy(aggS.at[pl.ds(NS * RPT, TAIL)],
                            agg_o.at[cid, pl.ds(NS * RPT, TAIL)])
            if with_deg:
                pltpu.sync_copy(degS.at[pl.ds(NS * RPT, TAIL)],
                                deg_o.at[cid, pl.ds(NS * RPT, TAIL)])

    return k


_SC_EDGE0 = _sc_edge_kernel(True)
_SC_EDGE1 = _sc_edge_kernel(False)


def _tc_enc_body(nf, we1, be1, we2, be2, w1s, w1d, b1, h_o, p_o, q_o):
    t = jnp.maximum(jnp.dot(nf[...], we1[...], preferred_element_type=_f32)
                    + be1[...], 0.0)
    h = jnp.dot(t, we2[...], preferred_element_type=_f32) + be2[...]
    h_o[...] = h
    p_o[...] = jnp.dot(h, w1s[...], preferred_element_type=_f32) + b1[...]
    q_o[...] = jnp.dot(h, w1d[...], preferred_element_type=_f32)


@jax.jit
def _tc_enc(nf, we1, be1, we2, be2, w1s, w1d, b1):
    return pl.pallas_call(
        _tc_enc_body,
        out_shape=[jax.ShapeDtypeStruct((N, H), _f32)] * 3,
    )(nf, we1, be1, we2, be2, w1s, w1d, b1)


def _tc_upd_body(has_next, h, ag, dg, w2, b2, wua, wub, bu1, wu2, bu2, *rest):
    aggp = ag[0] + ag[1]
    deg = dg[0] + dg[1]
    agg = jnp.dot(aggp, w2[...], preferred_element_type=_f32) + deg * b2[...]
    t = jnp.maximum(jnp.dot(h[...], wua[...], preferred_element_type=_f32)
                    + jnp.dot(agg, wub[...], preferred_element_type=_f32)
                    + bu1[...], 0.0)
    hn = h[...] + jnp.dot(t, wu2[...], preferred_element_type=_f32) + bu2[...]
    if has_next:
        w1s, w1d, b1, h_o, p_o, q_o = rest
        h_o[...] = hn
        p_o[...] = jnp.dot(hn, w1s[...], preferred_element_type=_f32) + b1[...]
        q_o[...] = jnp.dot(hn, w1d[...], preferred_element_type=_f32)
    else:
        wh1, bh1, wh2, bh2, out_o = rest
        u = jnp.maximum(jnp.dot(hn, wh1[...], preferred_element_type=_f32)
                        + bh1[...], 0.0)
        out_o[...] = jnp.dot(u, wh2[...], preferred_element_type=_f32) + bh2[...]


def _tc_upd(has_next, h, ag, dg, w2, b2, wua, wub, bu1, wu2, bu2, *rest):
    if has_next:
        out_shape = [jax.ShapeDtypeStruct((N, H), _f32)] * 3
    else:
        out_shape = [jax.ShapeDtypeStruct((N, 1), _f32)]
    return pl.pallas_call(
        functools.partial(_tc_upd_body, has_next),
        out_shape=out_shape,
    )(h, ag, dg, w2, b2, wua, wub, bu1, wu2, bu2, *rest)


def _row(b):
    return b.reshape(1, -1)


def kernel(node_features, edges, edge_features, params):
    src = edges[:, 0].reshape(NW, NCHUNK, NSUB, SUB)
    dst = edges[:, 1].reshape(NW, NCHUNK, NSUB, SUB)
    eft = edge_features.T

    ones80 = jnp.ones((SUB, 1), _f32)
    zrow = jnp.zeros((RPT, H), _f32)
    zcol = jnp.zeros((RPT, 1), _f32)

    enc, msg, upd, head = params["enc"], params["msg"], params["upd"], params["head"]

    msg_parts = []
    for r in range(2):
        w1 = msg[r][0]["W"]
        cw = jnp.zeros((8, H), _f32).at[0:3].set(w1[2 * H:2 * H + 3])
        msg_parts.append({
            "w1s": w1[0:H], "w1d": w1[H:2 * H], "cw": cw,
            "b1": _row(msg[r][0]["b"]),
            "w2": msg[r][1]["W"], "b2": _row(msg[r][1]["b"]),
        })

    h, p, q = _tc_enc(node_features, enc[0]["W"], _row(enc[0]["b"]),
                      enc[1]["W"], _row(enc[1]["b"]),
                      msg_parts[0]["w1s"], msg_parts[0]["w1d"],
                      msg_parts[0]["b1"])

    dg = None
    for r in range(2):
        mp = msg_parts[r]
        if r == 0:
            ag, dg = _SC_EDGE0(p, q, src, dst, eft, mp["cw"], ones80, zrow, zcol)
        else:
            ag, = _SC_EDGE1(p, q, src, dst, eft, mp["cw"], zrow)
        uw = upd[r][0]["W"]
        args = (h, ag, dg, mp["w2"], mp["b2"],
                uw[0:H], uw[H:2 * H], _row(upd[r][0]["b"]),
                upd[r][1]["W"], _row(upd[r][1]["b"]))
        if r == 0:
            nxt = msg_parts[1]
            h, p, q = _tc_upd(True, *args, nxt["w1s"], nxt["w1d"], nxt["b1"])
        else:
            out, = _tc_upd(False, *args, head[0]["W"], _row(head[0]["b"]),
                           head[1]["W"], _row(head[1]["b"]))
    return out[:, 0]
